# baseline (speedup 1.0000x reference)
import functools

import jax
import jax.numpy as jnp
from jax import lax
from jax.experimental import pallas as pl
from jax.experimental.pallas import tpu as pltpu
from jax.experimental.pallas import tpu_sc as plsc

L = 16
NC = 2
NS = 16
NW = NC * NS
CHUNK = 128
NBUF = 2


def _body(x_ref, seg_ref, extra_ref, gamma_ref, beta_ref, tok_ref, out_ref,
          idx_v, prow_v, seg_v, extra_v, gamma_v, beta_v,
          rows0, rows1, outb0, outb1,
          sem_g0, sem_g1, sem_o0, sem_o1,
          *, seq_len, n_seg, d):
    n_tok = x_ref.shape[0]
    jj = d // L
    per_w = n_tok // NW
    n_chunk = per_w // CHUNK
    orow_per_chunk = CHUNK * d // 128
    wid = lax.axis_index("s") * NC + lax.axis_index("c")
    base = wid * per_w
    obase = wid * (per_w * d // 128)

    pltpu.sync_copy(x_ref.at[pl.ds(base, per_w)], idx_v)
    pltpu.sync_copy(seg_ref.at[pl.ds(base, per_w)], seg_v)
    pltpu.sync_copy(extra_ref, extra_v)
    pltpu.sync_copy(gamma_ref, gamma_v)
    pltpu.sync_copy(beta_ref, beta_v)

    def shift_body(i, carry):
        prow_v[pl.ds(i * L, L)] = idx_v[pl.ds(i * L, L)] >> 1
        return carry
    lax.fori_loop(0, per_w // L, shift_body, 0)

    rows = (rows0, rows1)
    outb = (outb0, outb1)
    sem_g = (sem_g0, sem_g1)
    sem_o = (sem_o0, sem_o1)

    def gather_start(c, b):
        pltpu.async_copy(tok_ref.at[prow_v.at[pl.ds(c * CHUNK, CHUNK)]],
                         rows[b], sem_g[b])

    def gather_wait(b):
        pltpu.make_async_copy(tok_ref.at[prow_v.at[pl.ds(0, CHUNK)]],
                              rows[b], sem_g[b]).wait()

    def out_start(c, b):
        pltpu.async_copy(outb[b],
                         out_ref.at[pl.ds(obase + c * orow_per_chunk,
                                          orow_per_chunk)],
                         sem_o[b])

    def out_wait(b):
        pltpu.make_async_copy(outb[b],
                              out_ref.at[pl.ds(obase, orow_per_chunk)],
                              sem_o[b]).wait()

    for b in range(NBUF):
        gather_start(b, b)

    lane = lax.iota(jnp.int32, L)
    gvecs = [gamma_v[pl.ds(L * j, L)] for j in range(jj)]
    bvecs = [beta_v[pl.ds(L * j, L)] for j in range(jj)]

    def compute_chunk(c, rows_b, outb_b):
        def group_body(g, carry):
            goff = c * CHUNK + g * L
            p = (goff + lane) % seq_len
            s_vec = seg_v[pl.ds(goff, L)]
            scol_vec = s_vec * d
            hcol_vec = (idx_v[pl.ds(goff, L)] & 1) * d
            for tt in range(L):
                t = g * L + tt
                p_t = p[tt]
                sc_t = scol_vec[tt]
                hc_t = hcol_vec[tt]
                es = []
                for j in range(jj):
                    r = rows_b[t, pl.ds(hc_t + L * j, L)]
                    ex = extra_v[p_t, pl.ds(sc_t + L * j, L)]
                    es.append(r + ex)
                sum4 = es[0] + es[1] + es[2] + es[3]
                q = es[0] * es[0] + es[1] * es[1] + es[2] * es[2] + es[3] * es[3]
                ssum = jnp.sum(sum4)
                sq = jnp.sum(q)
                mean = ssum * (1.0 / d)
                var = sq * (1.0 / d) - mean * mean
                vv = var + 1e-5
                iv = lax.bitcast_convert_type(vv, jnp.int32)
                iv = jnp.int32(0x5F3759DF) - (iv >> 1)
                y = lax.bitcast_convert_type(iv, jnp.float32)
                for _ in range(3):
                    y = y * (1.5 - 0.5 * vv * y * y)
                sh = -mean * y
                orow = g * (L * d // 128) + (tt * d) // 128
                ocol = (tt * d) % 128
                for j in range(jj):
                    yv = es[j] * y + sh
                    yv = yv * gvecs[j] + bvecs[j]
                    outb_b[orow, pl.ds(ocol + L * j, L)] = yv
            return carry
        lax.fori_loop(0, CHUNK // L, group_body, 0)

    def outer(i, carry):
        for b in range(NBUF):
            c = i * NBUF + b

            @pl.when(c >= NBUF)
            def _():
                out_wait(b)

            gather_wait(b)
            compute_chunk(c, rows[b], outb[b])
            out_start(c, b)

            @pl.when(c + NBUF < n_chunk)
            def _():
                gather_start(c + NBUF, b)
        return carry

    lax.fori_loop(0, n_chunk // NBUF, outer, 0)
    for b in range(NBUF):
        out_wait(b)


def kernel(x, seg, tok_table, pos_table, seg_table, gamma, beta):
    bsz, s = x.shape
    v, d = tok_table.shape
    n_seg = seg_table.shape[0]
    n = bsz * s
    assert n % NW == 0
    per_w = n // NW
    assert per_w % CHUNK == 0 and per_w % s == 0
    assert (per_w // CHUNK) % NBUF == 0
    assert (CHUNK * d) % 128 == 0 and d % L == 0 and d // L == 4

    xf = x.reshape(n).astype(jnp.int32)
    sf = seg.reshape(n).astype(jnp.int32)
    extra = (pos_table[:s, None, :] + seg_table[None, :, :]).reshape(s, n_seg * d)
    tok2 = tok_table.reshape(v // 2, 2 * d)

    run = pl.kernel(
        functools.partial(_body, seq_len=s, n_seg=n_seg, d=d),
        out_type=jax.ShapeDtypeStruct((n * d // 128, 128), jnp.float32),
        mesh=plsc.VectorSubcoreMesh(core_axis_name="c", subcore_axis_name="s"),
        compiler_params=pltpu.CompilerParams(
            needs_layout_passes=False, use_tc_tiling_on_sc=True),
        scratch_types=[
            pltpu.VMEM((per_w,), jnp.int32),
            pltpu.VMEM((per_w,), jnp.int32),
            pltpu.VMEM((per_w,), jnp.int32),
            pltpu.VMEM((s, n_seg * d), jnp.float32),
            pltpu.VMEM((d,), jnp.float32),
            pltpu.VMEM((d,), jnp.float32),
            pltpu.VMEM((CHUNK, 2 * d), jnp.float32),
            pltpu.VMEM((CHUNK, 2 * d), jnp.float32),
            pltpu.VMEM((CHUNK * d // 128, 128), jnp.float32),
            pltpu.VMEM((CHUNK * d // 128, 128), jnp.float32),
            pltpu.SemaphoreType.DMA,
            pltpu.SemaphoreType.DMA,
            pltpu.SemaphoreType.DMA,
            pltpu.SemaphoreType.DMA,
        ],
    )
    out = run(xf, sf, extra, gamma, beta, tok2)
    return out.reshape(bsz, s, d)


# speedup vs baseline: 1.0089x; 1.0089x over previous
import functools

import jax
import jax.numpy as jnp
from jax import lax
from jax.experimental import pallas as pl
from jax.experimental.pallas import tpu as pltpu
from jax.experimental.pallas import tpu_sc as plsc

L = 16
NC = 2
NS = 16
NW = NC * NS
CHUNK = 128
NBUF = 2


def _body(x_ref, seg_ref, extra_ref, gamma_ref, beta_ref, tok_ref, out_ref,
          idx_v, seg_v, extra_v, gamma_v, beta_v,
          rows0, rows1, outb0, outb1,
          sem_g0, sem_g1, sem_o0, sem_o1,
          *, seq_len, n_seg, d):
    n_tok = x_ref.shape[0]
    jj = d // L
    per_w = n_tok // NW
    n_chunk = per_w // CHUNK
    wid = lax.axis_index("s") * NC + lax.axis_index("c")
    base = wid * per_w
    obase = wid * per_w * d

    pltpu.sync_copy(x_ref.at[pl.ds(base, per_w)], idx_v)
    pltpu.sync_copy(seg_ref.at[pl.ds(base, per_w)], seg_v)
    pltpu.sync_copy(extra_ref, extra_v)
    pltpu.sync_copy(gamma_ref, gamma_v)
    pltpu.sync_copy(beta_ref, beta_v)

    rows = (rows0, rows1)
    outb = (outb0, outb1)
    sem_g = (sem_g0, sem_g1)
    sem_o = (sem_o0, sem_o1)

    def gather_start(c, b):
        pltpu.async_copy(tok_ref.at[idx_v.at[pl.ds(c * CHUNK, CHUNK)]],
                         rows[b], sem_g[b])

    def gather_wait(b):
        pltpu.make_async_copy(tok_ref.at[idx_v.at[pl.ds(0, CHUNK)]],
                              rows[b], sem_g[b]).wait()

    def out_start(c, b):
        pltpu.async_copy(outb[b],
                         out_ref.at[pl.ds(obase + c * CHUNK * d, CHUNK * d)],
                         sem_o[b])

    def out_wait(b):
        pltpu.make_async_copy(outb[b],
                              out_ref.at[pl.ds(obase, CHUNK * d)],
                              sem_o[b]).wait()

    for b in range(NBUF):
        gather_start(b, b)

    lane = lax.iota(jnp.int32, L)
    gvecs = [gamma_v[pl.ds(L * j, L)] for j in range(jj)]
    bvecs = [beta_v[pl.ds(L * j, L)] for j in range(jj)]

    def compute_chunk(c, rows_b, outb_b):
        def group_body(g, carry):
            goff = c * CHUNK + g * L
            p = (goff + lane) % seq_len
            s_vec = seg_v[pl.ds(goff, L)]
            erow_vec = p * n_seg + s_vec
            for tt in range(L):
                t = g * L + tt
                erow = erow_vec[tt]
                es = []
                for j in range(jj):
                    r = rows_b[t, pl.ds(L * j, L)]
                    ex = extra_v[erow, pl.ds(L * j, L)]
                    es.append(r + ex)
                sum4 = es[0] + es[1] + es[2] + es[3]
                q = es[0] * es[0] + es[1] * es[1] + es[2] * es[2] + es[3] * es[3]
                ssum = jnp.sum(sum4)
                sq = jnp.sum(q)
                mean = ssum * (1.0 / d)
                var = sq * (1.0 / d) - mean * mean
                vv = var + 1e-5
                iv = lax.bitcast_convert_type(vv, jnp.int32)
                iv = jnp.int32(0x5F3759DF) - (iv >> 1)
                y = lax.bitcast_convert_type(iv, jnp.float32)
                for _ in range(3):
                    y = y * (1.5 - 0.5 * vv * y * y)
                sh = -mean * y
                for j in range(jj):
                    yv = es[j] * y + sh
                    yv = yv * gvecs[j] + bvecs[j]
                    outb_b[pl.ds(t * d + L * j, L)] = yv
            return carry
        lax.fori_loop(0, CHUNK // L, group_body, 0)

    def outer(i, carry):
        for b in range(NBUF):
            c = i * NBUF + b

            @pl.when(c >= NBUF)
            def _():
                out_wait(b)

            gather_wait(b)
            compute_chunk(c, rows[b], outb[b])
            out_start(c, b)

            @pl.when(c + NBUF < n_chunk)
            def _():
                gather_start(c + NBUF, b)
        return carry

    lax.fori_loop(0, n_chunk // NBUF, outer, 0)
    for b in range(NBUF):
        out_wait(b)


def kernel(x, seg, tok_table, pos_table, seg_table, gamma, beta):
    bsz, s = x.shape
    v, d = tok_table.shape
    n_seg = seg_table.shape[0]
    n = bsz * s
    assert n % NW == 0
    per_w = n // NW
    assert per_w % CHUNK == 0 and per_w % s == 0
    assert (per_w // CHUNK) % NBUF == 0
    assert (CHUNK * d) % 128 == 0 and d % L == 0 and d // L == 4

    xf = x.reshape(n).astype(jnp.int32)
    sf = seg.reshape(n).astype(jnp.int32)
    extra = (pos_table[:s, None, :] + seg_table[None, :, :]).reshape(s * n_seg, d)

    run = pl.kernel(
        functools.partial(_body, seq_len=s, n_seg=n_seg, d=d),
        out_type=jax.ShapeDtypeStruct((n * d,), jnp.float32),
        mesh=plsc.VectorSubcoreMesh(core_axis_name="c", subcore_axis_name="s"),
        compiler_params=pltpu.CompilerParams(
            needs_layout_passes=False, use_tc_tiling_on_sc=False),
        scratch_types=[
            pltpu.VMEM((per_w,), jnp.int32),
            pltpu.VMEM((per_w,), jnp.int32),
            pltpu.VMEM((s * n_seg, d), jnp.float32),
            pltpu.VMEM((d,), jnp.float32),
            pltpu.VMEM((d,), jnp.float32),
            pltpu.VMEM((CHUNK, d), jnp.float32),
            pltpu.VMEM((CHUNK, d), jnp.float32),
            pltpu.VMEM((CHUNK * d,), jnp.float32),
            pltpu.VMEM((CHUNK * d,), jnp.float32),
            pltpu.SemaphoreType.DMA,
            pltpu.SemaphoreType.DMA,
            pltpu.SemaphoreType.DMA,
            pltpu.SemaphoreType.DMA,
        ],
    )
    out = run(xf, sf, extra, gamma, beta, tok_table)
    return out.reshape(bsz, s, d)


# speedup vs baseline: 1.1937x; 1.1832x over previous
import functools

import jax
import jax.numpy as jnp
from jax import lax
from jax.experimental import pallas as pl
from jax.experimental.pallas import tpu as pltpu
from jax.experimental.pallas import tpu_sc as plsc

L = 16
NC = 2
NS = 16
NW = NC * NS
CHUNK = 128
NBUF = 2


_GDN = lax.GatherDimensionNumbers(
    offset_dims=(), collapsed_slice_dims=(0,), start_index_map=(0,))


def _take16(v, idx):
    return lax.gather(v, idx[:, None], _GDN, (1,),
                      mode=lax.GatherScatterMode.PROMISE_IN_BOUNDS)


def _body(x_ref, seg_ref, extra_ref, gamma_ref, beta_ref, tok_ref, out_ref,
          idx_v, seg_v, extra_v, gamma_v, beta_v,
          rows0, rows1, outb0, outb1,
          sem_g0, sem_g1, sem_o0, sem_o1,
          *, seq_len, n_seg, d):
    n_tok = x_ref.shape[0]
    jj = d // L
    per_w = n_tok // NW
    n_chunk = per_w // CHUNK
    wid = lax.axis_index("s") * NC + lax.axis_index("c")
    base = wid * per_w
    obase = wid * per_w * d

    pltpu.sync_copy(x_ref.at[pl.ds(base, per_w)], idx_v)
    pltpu.sync_copy(seg_ref.at[pl.ds(base, per_w)], seg_v)
    pltpu.sync_copy(extra_ref, extra_v)
    pltpu.sync_copy(gamma_ref, gamma_v)
    pltpu.sync_copy(beta_ref, beta_v)

    rows = (rows0, rows1)
    outb = (outb0, outb1)
    sem_g = (sem_g0, sem_g1)
    sem_o = (sem_o0, sem_o1)

    def gather_start(c, b):
        pltpu.async_copy(tok_ref.at[idx_v.at[pl.ds(c * CHUNK, CHUNK)]],
                         rows[b], sem_g[b])

    def gather_wait(b):
        pltpu.make_async_copy(tok_ref.at[idx_v.at[pl.ds(0, CHUNK)]],
                              rows[b], sem_g[b]).wait()

    def out_start(c, b):
        pltpu.async_copy(outb[b],
                         out_ref.at[pl.ds(obase + c * CHUNK * d, CHUNK * d)],
                         sem_o[b])

    def out_wait(b):
        pltpu.make_async_copy(outb[b],
                              out_ref.at[pl.ds(obase, CHUNK * d)],
                              sem_o[b]).wait()

    for b in range(NBUF):
        gather_start(b, b)

    lane = lax.iota(jnp.int32, L)
    gvecs = [gamma_v[pl.ds(L * j, L)] for j in range(jj)]
    bvecs = [beta_v[pl.ds(L * j, L)] for j in range(jj)]

    xor_idx = [lane ^ stp for stp in (1, 2, 4, 8)]
    splat_idx = [jnp.full((L,), tt, jnp.int32) for tt in range(L)]

    def compute_chunk(c, rows_b, outb_b):
        def group_body(g, carry):
            goff = c * CHUNK + g * L
            p = (goff + lane) % seq_len
            s_vec = seg_v[pl.ds(goff, L)]
            erow_vec = p * n_seg + s_vec
            sums = jnp.zeros((L,), jnp.float32)
            sqs = jnp.zeros((L,), jnp.float32)
            for tt in range(L):
                t = g * L + tt
                erow = erow_vec[tt]
                es = []
                for j in range(jj):
                    r = rows_b[t, pl.ds(L * j, L)]
                    ex = extra_v[erow, pl.ds(L * j, L)]
                    e = r + ex
                    outb_b[pl.ds(t * d + L * j, L)] = e
                    es.append(e)
                part = es[0] + es[1] + es[2] + es[3]
                q = es[0] * es[0] + es[1] * es[1] + es[2] * es[2] + es[3] * es[3]
                for xi in xor_idx:
                    part = part + _take16(part, xi)
                    q = q + _take16(q, xi)
                onehot = lane == tt
                sums = jnp.where(onehot, part, sums)
                sqs = jnp.where(onehot, q, sqs)
            mean = sums * (1.0 / d)
            var = sqs * (1.0 / d) - mean * mean
            vv = var + 1e-5
            iv = plsc.bitcast(vv, jnp.int32)
            iv = jnp.int32(0x5F3759DF) - (iv >> 1)
            y = plsc.bitcast(iv, jnp.float32)
            for _ in range(3):
                y = y * (1.5 - 0.5 * vv * y * y)
            sh = -mean * y
            for tt in range(L):
                t = g * L + tt
                ysp = _take16(y, splat_idx[tt])
                ssp = _take16(sh, splat_idx[tt])
                for j in range(jj):
                    e = outb_b[pl.ds(t * d + L * j, L)]
                    yv = e * ysp + ssp
                    yv = yv * gvecs[j] + bvecs[j]
                    outb_b[pl.ds(t * d + L * j, L)] = yv
            return carry
        lax.fori_loop(0, CHUNK // L, group_body, 0)

    def outer(i, carry):
        for b in range(NBUF):
            c = i * NBUF + b

            @pl.when(c >= NBUF)
            def _():
                out_wait(b)

            gather_wait(b)
            compute_chunk(c, rows[b], outb[b])
            out_start(c, b)

            @pl.when(c + NBUF < n_chunk)
            def _():
                gather_start(c + NBUF, b)
        return carry

    lax.fori_loop(0, n_chunk // NBUF, outer, 0)
    for b in range(NBUF):
        out_wait(b)


def kernel(x, seg, tok_table, pos_table, seg_table, gamma, beta):
    bsz, s = x.shape
    v, d = tok_table.shape
    n_seg = seg_table.shape[0]
    n = bsz * s
    assert n % NW == 0
    per_w = n // NW
    assert per_w % CHUNK == 0 and per_w % s == 0
    assert (per_w // CHUNK) % NBUF == 0
    assert (CHUNK * d) % 128 == 0 and d % L == 0 and d // L == 4

    xf = x.reshape(n).astype(jnp.int32)
    sf = seg.reshape(n).astype(jnp.int32)
    extra = (pos_table[:s, None, :] + seg_table[None, :, :]).reshape(s * n_seg, d)

    run = pl.kernel(
        functools.partial(_body, seq_len=s, n_seg=n_seg, d=d),
        out_type=jax.ShapeDtypeStruct((n * d,), jnp.float32),
        mesh=plsc.VectorSubcoreMesh(core_axis_name="c", subcore_axis_name="s"),
        compiler_params=pltpu.CompilerParams(
            needs_layout_passes=False, use_tc_tiling_on_sc=False),
        scratch_types=[
            pltpu.VMEM((per_w,), jnp.int32),
            pltpu.VMEM((per_w,), jnp.int32),
            pltpu.VMEM((s * n_seg, d), jnp.float32),
            pltpu.VMEM((d,), jnp.float32),
            pltpu.VMEM((d,), jnp.float32),
            pltpu.VMEM((CHUNK, d), jnp.float32),
            pltpu.VMEM((CHUNK, d), jnp.float32),
            pltpu.VMEM((CHUNK * d,), jnp.float32),
            pltpu.VMEM((CHUNK * d,), jnp.float32),
            pltpu.SemaphoreType.DMA,
            pltpu.SemaphoreType.DMA,
            pltpu.SemaphoreType.DMA,
            pltpu.SemaphoreType.DMA,
        ],
    )
    out = run(xf, sf, extra, gamma, beta, tok_table)
    return out.reshape(bsz, s, d)
